# Initial kernel scaffold; baseline (speedup 1.0000x reference)
import jax
import jax.numpy as jnp
from jax.experimental import pallas as pl


def kernel(x, edge_index, Wl1, bl1, Wr1, Wl2, bl2, Wr2):
    raise NotImplementedError("write your pallas kernel here")


# speedup vs baseline: 3.9979x; 3.9979x over previous
import functools

import jax
import jax.numpy as jnp
from jax import lax
from jax.experimental import pallas as pl
from jax.experimental.pallas import tpu as pltpu
from jax.experimental.pallas import tpu_sc as plsc

N = 10000
D = 128
NC, NS = 2, 16
NW = NC * NS
CHUNK = 128
N_PAD = 10240
ROWS_PER_SUB = N_PAD // NS


def _sc_aggregate(table, src_pad, dst_pad, zeros2d, zeros1d, ones1d,
                  with_counts):
  e_pad = src_pad.shape[0]
  epw = e_pad // NW
  n_chunks = epw // CHUNK
  mesh = plsc.VectorSubcoreMesh(core_axis_name="c", subcore_axis_name="s")

  out_type = [jax.ShapeDtypeStruct((NC * N_PAD, D), jnp.float32)]
  if with_counts:
    out_type.append(jax.ShapeDtypeStruct((NC * N_PAD,), jnp.float32))

  scratch = [
      pltpu.VMEM_SHARED((N_PAD, D), jnp.float32),
      pltpu.VMEM_SHARED((N_PAD,), jnp.float32),
      pltpu.VMEM((CHUNK,), jnp.int32),
      pltpu.VMEM((CHUNK,), jnp.int32),
      pltpu.VMEM((CHUNK, D), jnp.float32),
      pltpu.VMEM((CHUNK,), jnp.float32),
      pltpu.SemaphoreType.DMA,
  ]

  def body(table_h, src_h, dst_h, z2_h, z1_h, ones_h, *rest):
    if with_counts:
      sums_out, cnt_out = rest[0], rest[1]
      rest = rest[2:]
    else:
      sums_out, cnt_out = rest[0], None
      rest = rest[1:]
    acc, cnt_acc, src_v, dst_v, rows_v, ones_v, sem = rest

    cid = lax.axis_index("c")
    sid = lax.axis_index("s")
    wid = cid * NS + sid
    stripe = sid * ROWS_PER_SUB

    pltpu.sync_copy(z2_h, acc.at[pl.ds(stripe, ROWS_PER_SUB)])
    pltpu.sync_copy(z1_h, cnt_acc.at[pl.ds(stripe, ROWS_PER_SUB)])
    pltpu.sync_copy(ones_h, ones_v)
    plsc.subcore_barrier()

    base = wid * epw

    @pl.loop(0, n_chunks)
    def _chunk(c):
      off = base + c * CHUNK
      pltpu.sync_copy(src_h.at[pl.ds(off, CHUNK)], src_v)
      pltpu.sync_copy(dst_h.at[pl.ds(off, CHUNK)], dst_v)
      pltpu.async_copy(table_h.at[src_v], rows_v, sem).wait()
      pltpu.sync_copy(rows_v, acc.at[dst_v], add=True)
      if with_counts:
        pltpu.sync_copy(ones_v, cnt_acc.at[dst_v], add=True)

    plsc.subcore_barrier()
    out_off = cid * N_PAD + stripe
    pltpu.sync_copy(acc.at[pl.ds(stripe, ROWS_PER_SUB)],
                    sums_out.at[pl.ds(out_off, ROWS_PER_SUB)])
    if with_counts:
      pltpu.sync_copy(cnt_acc.at[pl.ds(stripe, ROWS_PER_SUB)],
                      cnt_out.at[pl.ds(out_off, ROWS_PER_SUB)])

  fn = pl.kernel(body, out_type=tuple(out_type), mesh=mesh,
                 scratch_types=scratch)
  return fn(table, src_pad, dst_pad, zeros2d, zeros1d, ones1d)


def _dense_body(s0_ref, s1_ref, c_ref, x_ref, wl_ref, bl_ref, wr_ref, o_ref):
  c = c_ref[:, 0] + c_ref[:, 1]
  inv = 1.0 / jnp.maximum(c, 1.0)
  mean = (s0_ref[...] + s1_ref[...]) * inv[:, None]
  out = (jnp.dot(mean, wl_ref[...], preferred_element_type=jnp.float32)
         + jnp.dot(x_ref[...], wr_ref[...], preferred_element_type=jnp.float32)
         + bl_ref[...])
  nrm = jnp.sqrt(jnp.sum(out * out, axis=-1, keepdims=True))
  out = out / jnp.maximum(nrm, 1e-12)
  o_ref[...] = jnp.where(out > 0, out, jnp.exp(out) - 1.0)


def _tc_dense(s0, s1, cpair, x, wl, bl, wr):
  rows = 1000
  grid = (N // rows,)
  return pl.pallas_call(
      _dense_body,
      grid=grid,
      in_specs=[
          pl.BlockSpec((rows, D), lambda i: (i, 0)),
          pl.BlockSpec((rows, D), lambda i: (i, 0)),
          pl.BlockSpec((rows, 2), lambda i: (i, 0)),
          pl.BlockSpec((rows, D), lambda i: (i, 0)),
          pl.BlockSpec((D, D), lambda i: (0, 0)),
          pl.BlockSpec((1, D), lambda i: (0, 0)),
          pl.BlockSpec((D, D), lambda i: (0, 0)),
      ],
      out_specs=pl.BlockSpec((rows, D), lambda i: (i, 0)),
      out_shape=jax.ShapeDtypeStruct((N, D), jnp.float32),
  )(s0, s1, cpair, x, wl, bl, wr)


def kernel(x, edge_index, Wl1, bl1, Wr1, Wl2, bl2, Wr2):
  src = edge_index[0]
  dst = edge_index[1]
  e = src.shape[0]
  n_chunks = -(-e // (NW * CHUNK))
  e_pad = NW * CHUNK * n_chunks
  pad = e_pad - e
  src_p = jnp.concatenate([src, jnp.zeros((pad,), jnp.int32)])
  dst_p = jnp.concatenate([dst, jnp.full((pad,), N, jnp.int32)])
  z2 = jnp.zeros((ROWS_PER_SUB, D), jnp.float32)
  z1 = jnp.zeros((ROWS_PER_SUB,), jnp.float32)
  ones = jnp.ones((CHUNK,), jnp.float32)
  bl1r = bl1.reshape(1, D)
  bl2r = bl2.reshape(1, D)

  sums1, cnt = _sc_aggregate(x, src_p, dst_p, z2, z1, ones, True)
  cpair = jnp.stack([cnt[:N], cnt[N_PAD:N_PAD + N]], axis=1)
  h1 = _tc_dense(sums1[:N], sums1[N_PAD:N_PAD + N], cpair, x, Wl1, bl1r, Wr1)

  (sums2,) = _sc_aggregate(h1, src_p, dst_p, z2, z1, ones, False)
  h2 = _tc_dense(sums2[:N], sums2[N_PAD:N_PAD + N], cpair, h1, Wl2, bl2r, Wr2)
  return h2
